# baseline (speedup 1.0000x reference)
import functools

import jax
import jax.numpy as jnp
from jax import lax
from jax.experimental import pallas as pl
from jax.experimental.pallas import tpu as pltpu
from jax.experimental.pallas import tpu_sc as plsc

B, N, D = 8, 4096, 256
LOG_N = 12


_CW = 1024
_NCHK = N // _CW
_LOG_CW = 10


def _roll_l(x, s):
    return jnp.concatenate([x[:, s:], x[:, :s]], axis=1)


def _roll_r(x, s):
    return jnp.concatenate([x[:, -s:], x[:, :-s]], axis=1)


def _pxor(x, s):
    w = x.shape[1]
    pieces = []
    for base in range(0, w, 2 * s):
        pieces.append(x[:, base + s:base + 2 * s])
        pieces.append(x[:, base:base + s])
    return jnp.concatenate(pieces, axis=1)


def _local_stage(v, ix, col, j, desc):
    s = 1 << j
    upper = (col & s) != 0
    if s >= 128:
        pv = _pxor(v, s)
        pi = _pxor(ix, s)
    else:
        pv = jnp.where(upper, _roll_r(v, s), _roll_l(v, s))
        pi = jnp.where(upper, _roll_r(ix, s), _roll_l(ix, s))
    gt = (v > pv) | ((v == pv) & (ix > pi))
    if isinstance(desc, bool):
        take_self = (gt ^ upper) if desc else ~(gt ^ upper)
    else:
        take_self = ~(gt ^ upper ^ desc)
    return jnp.where(take_self, v, pv), jnp.where(take_self, ix, pi)


def _cross_stage(a, b, desc):
    va, ia = a
    vb, ib = b
    gt = (va > vb) | ((va == vb) & (ia > ib))
    swap = ~gt if desc else gt
    nva = jnp.where(swap, vb, va)
    nvb = jnp.where(swap, va, vb)
    nia = jnp.where(swap, ib, ia)
    nib = jnp.where(swap, ia, ib)
    return (nva, nia), (nvb, nib)


def _sort_body(phi_ref, sphi_ref, idx_ref):
    base = lax.broadcasted_iota(jnp.int32, (B, _CW), 1)
    cols = [base + c * _CW for c in range(_NCHK)]
    vs = [phi_ref[:, c * _CW:(c + 1) * _CW] for c in range(_NCHK)]
    ixs = list(cols)

    for k in range(_LOG_CW):
        for j in range(k, -1, -1):
            for c in range(_NCHK):
                if k + 1 < _LOG_CW:
                    desc = ((cols[c] >> (k + 1)) & 1) == 1
                else:
                    desc = bool(c & 1)
                vs[c], ixs[c] = _local_stage(vs[c], ixs[c], cols[c], j, desc)

    ch = [(vs[c], ixs[c]) for c in range(_NCHK)]

    ch[0], ch[1] = _cross_stage(ch[0], ch[1], False)
    ch[2], ch[3] = _cross_stage(ch[2], ch[3], True)
    for j in range(_LOG_CW - 1, -1, -1):
        for c in range(_NCHK):
            v, ix = _local_stage(*ch[c], cols[c], j, bool(c & 2))
            ch[c] = (v, ix)

    ch[0], ch[2] = _cross_stage(ch[0], ch[2], False)
    ch[1], ch[3] = _cross_stage(ch[1], ch[3], False)
    ch[0], ch[1] = _cross_stage(ch[0], ch[1], False)
    ch[2], ch[3] = _cross_stage(ch[2], ch[3], False)
    for j in range(_LOG_CW - 1, -1, -1):
        for c in range(_NCHK):
            v, ix = _local_stage(*ch[c], cols[c], j, False)
            ch[c] = (v, ix)

    row = lax.broadcasted_iota(jnp.int32, (B, _CW), 0)
    for c in range(_NCHK):
        v, ix = ch[c]
        sphi_ref[:, c * _CW:(c + 1) * _CW] = v
        idx_ref[:, c * _CW:(c + 1) * _CW] = ix + row * N


def _argsort_batch(phi):
    return pl.pallas_call(
        _sort_body,
        out_shape=(
            jax.ShapeDtypeStruct((B, N), jnp.float32),
            jax.ShapeDtypeStruct((B, N), jnp.int32),
        ),
    )(phi)


_NC, _NS = 2, 16
_NW = _NC * _NS
_ROWS_PER_W = (B * N) // _NW
_CH = 32
_NCHUNK = _ROWS_PER_W // _CH
_NBUF = 14


def _gather_kernel(tab_hbm, idx_hbm, out_hbm, idx_v, bufs, gsems, wsems):
    wid = lax.axis_index("s") * _NC + lax.axis_index("c")
    pltpu.sync_copy(
        idx_hbm.at[wid // 4, pl.ds((wid % 4) * _ROWS_PER_W, _ROWS_PER_W)],
        idx_v)

    def gather(c):
        b = c % _NBUF
        return pltpu.async_copy(tab_hbm.at[idx_v.at[pl.ds(c * _CH, _CH)]],
                                bufs[b], gsems[b])

    def write(c):
        b = c % _NBUF
        dst = out_hbm.at[pl.ds(wid * _ROWS_PER_W + c * _CH, _CH)]
        return pltpu.async_copy(bufs[b], dst, wsems[b])

    depth = 8
    gh = [None] * _NBUF
    wh = [None] * _NBUF
    for c in range(_NCHUNK):
        b = c % _NBUF
        if wh[b] is not None:
            wh[b].wait()
            wh[b] = None
        gh[b] = gather(c)
        if c >= depth:
            bp = (c - depth) % _NBUF
            gh[bp].wait()
            wh[bp] = write(c - depth)
    for c in range(max(0, _NCHUNK - depth), _NCHUNK):
        b = c % _NBUF
        gh[b].wait()
        wh[b] = write(c)
    for b in range(_NBUF):
        if wh[b] is not None:
            wh[b].wait()


@functools.cache
def _make_gather_rows():
    @functools.partial(
        pl.kernel,
        mesh=plsc.VectorSubcoreMesh(core_axis_name="c", subcore_axis_name="s"),
        out_type=jax.ShapeDtypeStruct((B * N, D), jnp.float32),
        scratch_types=[
            pltpu.VMEM((_ROWS_PER_W,), jnp.int32),
        ] + [pltpu.VMEM((_CH, D), jnp.float32)] * _NBUF
          + [pltpu.SemaphoreType.DMA] * (2 * _NBUF),
    )
    def _gather_rows(tab_hbm, idx_hbm, out_hbm, idx_v, *rest):
        bufs = rest[:_NBUF]
        gsems = rest[_NBUF:2 * _NBUF]
        wsems = rest[2 * _NBUF:]
        _gather_kernel(tab_hbm, idx_hbm, out_hbm, idx_v, bufs, gsems, wsems)

    return _gather_rows


def kernel(hit_embed, hit_phi, key_embed, key_phi):
    gather = _make_gather_rows()
    hit_sphi, hit_idx = _argsort_batch(hit_phi)
    hit_s = gather(hit_embed.reshape(B * N, D), hit_idx)
    key_sphi, key_idx = _argsort_batch(key_phi)
    key_s = gather(key_embed.reshape(B * N, D), key_idx)
    return (
        hit_s.reshape(B, N, D),
        hit_sphi,
        key_s.reshape(B, N, D),
        key_sphi,
    )


# speedup vs baseline: 1.0244x; 1.0244x over previous
import functools

import jax
import jax.numpy as jnp
from jax import lax
from jax.experimental import pallas as pl
from jax.experimental.pallas import tpu as pltpu
from jax.experimental.pallas import tpu_sc as plsc

B, N, D = 8, 4096, 256
LOG_N = 12


_CW = 1024
_NCHK = N // _CW
_LOG_CW = 10


def _roll_l(x, s):
    return jnp.concatenate([x[:, s:], x[:, :s]], axis=1)


def _roll_r(x, s):
    return jnp.concatenate([x[:, -s:], x[:, :-s]], axis=1)


def _pxor(x, s):
    w = x.shape[1]
    pieces = []
    for base in range(0, w, 2 * s):
        pieces.append(x[:, base + s:base + 2 * s])
        pieces.append(x[:, base:base + s])
    return jnp.concatenate(pieces, axis=1)


def _local_stage(v, ix, col, j, desc):
    s = 1 << j
    upper = (col & s) != 0
    if s >= 128:
        pv = _pxor(v, s)
        pi = _pxor(ix, s)
    else:
        pv = jnp.where(upper, _roll_r(v, s), _roll_l(v, s))
        pi = jnp.where(upper, _roll_r(ix, s), _roll_l(ix, s))
    gt = (v > pv) | ((v == pv) & (ix > pi))
    if isinstance(desc, bool):
        take_self = (gt ^ upper) if desc else ~(gt ^ upper)
    else:
        take_self = ~(gt ^ upper ^ desc)
    return jnp.where(take_self, v, pv), jnp.where(take_self, ix, pi)


def _cross_stage(a, b, desc):
    va, ia = a
    vb, ib = b
    gt = (va > vb) | ((va == vb) & (ia > ib))
    swap = ~gt if desc else gt
    nva = jnp.where(swap, vb, va)
    nvb = jnp.where(swap, va, vb)
    nia = jnp.where(swap, ib, ia)
    nib = jnp.where(swap, ia, ib)
    return (nva, nia), (nvb, nib)


def _sort_body(phi_ref, sphi_ref, idx_ref):
    base = lax.broadcasted_iota(jnp.int32, (B, _CW), 1)
    cols = [base + c * _CW for c in range(_NCHK)]
    vs = [phi_ref[:, c * _CW:(c + 1) * _CW] for c in range(_NCHK)]
    ixs = list(cols)

    for k in range(_LOG_CW):
        for j in range(k, -1, -1):
            for c in range(_NCHK):
                if k + 1 < _LOG_CW:
                    desc = ((cols[c] >> (k + 1)) & 1) == 1
                else:
                    desc = bool(c & 1)
                vs[c], ixs[c] = _local_stage(vs[c], ixs[c], cols[c], j, desc)

    ch = [(vs[c], ixs[c]) for c in range(_NCHK)]

    ch[0], ch[1] = _cross_stage(ch[0], ch[1], False)
    ch[2], ch[3] = _cross_stage(ch[2], ch[3], True)
    for j in range(_LOG_CW - 1, -1, -1):
        for c in range(_NCHK):
            v, ix = _local_stage(*ch[c], cols[c], j, bool(c & 2))
            ch[c] = (v, ix)

    ch[0], ch[2] = _cross_stage(ch[0], ch[2], False)
    ch[1], ch[3] = _cross_stage(ch[1], ch[3], False)
    ch[0], ch[1] = _cross_stage(ch[0], ch[1], False)
    ch[2], ch[3] = _cross_stage(ch[2], ch[3], False)
    for j in range(_LOG_CW - 1, -1, -1):
        for c in range(_NCHK):
            v, ix = _local_stage(*ch[c], cols[c], j, False)
            ch[c] = (v, ix)

    row = lax.broadcasted_iota(jnp.int32, (B, _CW), 0)
    for c in range(_NCHK):
        v, ix = ch[c]
        sphi_ref[:, c * _CW:(c + 1) * _CW] = v
        idx_ref[:, c * _CW:(c + 1) * _CW] = ix + row * N


def _argsort_batch(phi):
    return pl.pallas_call(
        _sort_body,
        out_shape=(
            jax.ShapeDtypeStruct((B, N), jnp.float32),
            jax.ShapeDtypeStruct((B, N), jnp.int32),
        ),
    )(phi)


_NC, _NS = 2, 16
_NW = _NC * _NS
_ROWS_PER_W = (B * N) // _NW
_CH = 64
_NCHUNK = _ROWS_PER_W // _CH
_NBUF = 7


def _gather_kernel(tab_hbm, idx_hbm, out_hbm, idx_v, bufs, gsems, wsems):
    wid = lax.axis_index("s") * _NC + lax.axis_index("c")
    pltpu.sync_copy(
        idx_hbm.at[wid // 4, pl.ds((wid % 4) * _ROWS_PER_W, _ROWS_PER_W)],
        idx_v)

    def gather(c):
        b = c % _NBUF
        return pltpu.async_copy(tab_hbm.at[idx_v.at[pl.ds(c * _CH, _CH)]],
                                bufs[b], gsems[b])

    def write(c):
        b = c % _NBUF
        dst = out_hbm.at[pl.ds(wid * _ROWS_PER_W + c * _CH, _CH)]
        return pltpu.async_copy(bufs[b], dst, wsems[b])

    depth = 5
    gh = [None] * _NBUF
    wh = [None] * _NBUF
    for c in range(_NCHUNK):
        b = c % _NBUF
        if wh[b] is not None:
            wh[b].wait()
            wh[b] = None
        gh[b] = gather(c)
        if c >= depth:
            bp = (c - depth) % _NBUF
            gh[bp].wait()
            wh[bp] = write(c - depth)
    for c in range(max(0, _NCHUNK - depth), _NCHUNK):
        b = c % _NBUF
        gh[b].wait()
        wh[b] = write(c)
    for b in range(_NBUF):
        if wh[b] is not None:
            wh[b].wait()


@functools.cache
def _make_gather_rows():
    @functools.partial(
        pl.kernel,
        mesh=plsc.VectorSubcoreMesh(core_axis_name="c", subcore_axis_name="s"),
        out_type=jax.ShapeDtypeStruct((B * N, D), jnp.float32),
        scratch_types=[
            pltpu.VMEM((_ROWS_PER_W,), jnp.int32),
        ] + [pltpu.VMEM((_CH, D), jnp.float32)] * _NBUF
          + [pltpu.SemaphoreType.DMA] * (2 * _NBUF),
    )
    def _gather_rows(tab_hbm, idx_hbm, out_hbm, idx_v, *rest):
        bufs = rest[:_NBUF]
        gsems = rest[_NBUF:2 * _NBUF]
        wsems = rest[2 * _NBUF:]
        _gather_kernel(tab_hbm, idx_hbm, out_hbm, idx_v, bufs, gsems, wsems)

    return _gather_rows


def kernel(hit_embed, hit_phi, key_embed, key_phi):
    gather = _make_gather_rows()
    hit_sphi, hit_idx = _argsort_batch(hit_phi)
    hit_s = gather(hit_embed.reshape(B * N, D), hit_idx)
    key_sphi, key_idx = _argsort_batch(key_phi)
    key_s = gather(key_embed.reshape(B * N, D), key_idx)
    return (
        hit_s.reshape(B, N, D),
        hit_sphi,
        key_s.reshape(B, N, D),
        key_sphi,
    )
